# baseline (device time: 228475 ns/iter reference)
import jax
import jax.numpy as jnp
from jax import lax
from jax.experimental import pallas as pl
from jax.experimental.pallas import tpu as pltpu

T = 2048
D = 4096
V_SHARD = 8192
BV = 256
NBLK = V_SHARD // BV


def kernel(x, W, labels):
    labels2d = labels.reshape(T, 1)

    def body(x_ref, w_ref, lbl_ref, out_ref,
             eacc, lacc, s_loc, l_loc, s_rx, l_rx,
             send_sems, recv_sems, ack_sem):
        j = pl.program_id(0)
        my_x = lax.axis_index("x")
        my_y = lax.axis_index("y")
        my_z = lax.axis_index("z")

        logits = jnp.dot(x_ref[...], w_ref[...],
                         preferred_element_type=jnp.float32,
                         precision=lax.Precision.DEFAULT)

        offset = my_x * V_SHARD + j * BV
        idx = lbl_ref[...] - offset
        col = lax.broadcasted_iota(jnp.int32, (T, BV), 1)
        contrib = jnp.where(col == idx, logits, 0.0)
        e = jnp.exp(logits)

        @pl.when(j == 0)
        def _():
            eacc[...] = e
            lacc[...] = contrib

        @pl.when(j > 0)
        def _():
            eacc[...] = eacc[...] + e
            lacc[...] = lacc[...] + contrib

        @pl.when(j == NBLK - 1)
        def _():
            s_loc[...] = jnp.sum(eacc[...], axis=1, keepdims=True)
            l_loc[...] = jnp.sum(lacc[...], axis=1, keepdims=True)

            partner = (1 - my_x, my_y, my_z)
            copies = []
            for k, (src, dst) in enumerate(((s_loc, s_rx), (l_loc, l_rx))):
                c = pltpu.make_async_remote_copy(
                    src_ref=src, dst_ref=dst,
                    send_sem=send_sems.at[k], recv_sem=recv_sems.at[k],
                    device_id=partner,
                    device_id_type=pl.DeviceIdType.MESH)
                c.start()
                copies.append(c)
            for c in copies:
                c.wait()

            out_ref[...] = (jnp.log(s_loc[...] + s_rx[...])
                            - (l_loc[...] + l_rx[...]))

            pl.semaphore_signal(ack_sem, 1, device_id=partner,
                                device_id_type=pl.DeviceIdType.MESH)
            pl.semaphore_wait(ack_sem, 1)

    out = pl.pallas_call(
        body,
        grid=(NBLK,),
        in_specs=[
            pl.BlockSpec((T, D), lambda j: (0, 0)),
            pl.BlockSpec((D, BV), lambda j: (0, j)),
            pl.BlockSpec((T, 1), lambda j: (0, 0)),
        ],
        out_specs=pl.BlockSpec((T, 1), lambda j: (0, 0)),
        out_shape=jax.ShapeDtypeStruct((T, 1), jnp.float32),
        scratch_shapes=[
            pltpu.VMEM((T, BV), jnp.float32),
            pltpu.VMEM((T, BV), jnp.float32),
            pltpu.VMEM((T, 1), jnp.float32),
            pltpu.VMEM((T, 1), jnp.float32),
            pltpu.VMEM((T, 1), jnp.float32),
            pltpu.VMEM((T, 1), jnp.float32),
            pltpu.SemaphoreType.DMA((2,)),
            pltpu.SemaphoreType.DMA((2,)),
            pltpu.SemaphoreType.REGULAR,
        ],
        compiler_params=pltpu.CompilerParams(
            dimension_semantics=("arbitrary",),
            vmem_limit_bytes=64 * 1024 * 1024,
        ),
    )(x, W, labels2d)
    return out.reshape(T)


# device time: 227516 ns/iter; 1.0042x vs baseline; 1.0042x over previous
import jax
import jax.numpy as jnp
from jax import lax
from jax.experimental import pallas as pl
from jax.experimental.pallas import tpu as pltpu

T = 2048
D = 4096
V_SHARD = 8192
BV = 256
NBLK = V_SHARD // BV


def kernel(x, W, labels):
    labels2d = labels.reshape(T, 1)

    def body(x_ref, w_ref, lbl_ref, out_ref,
             eacc, lacc, s_loc, l_loc, s_rx, l_rx,
             send_sems, recv_sems, ack_sem):
        j = pl.program_id(0)
        my_x = lax.axis_index("x")
        my_y = lax.axis_index("y")
        my_z = lax.axis_index("z")

        w_bf = w_ref[...].astype(jnp.bfloat16)
        logits = jnp.dot(x_ref[...].astype(jnp.bfloat16), w_bf,
                         preferred_element_type=jnp.float32)

        offset = my_x * V_SHARD + j * BV
        idx = lbl_ref[...] - offset
        col = lax.broadcasted_iota(jnp.int32, (T, BV), 1)
        contrib = jnp.where(col == idx, logits, 0.0)
        e = jnp.exp(logits)

        @pl.when(j == 0)
        def _():
            eacc[...] = e
            lacc[...] = contrib

        @pl.when(j > 0)
        def _():
            eacc[...] = eacc[...] + e
            lacc[...] = lacc[...] + contrib

        @pl.when(j == NBLK - 1)
        def _():
            s_loc[...] = jnp.sum(eacc[...], axis=1, keepdims=True)
            l_loc[...] = jnp.sum(lacc[...], axis=1, keepdims=True)

            partner = (1 - my_x, my_y, my_z)
            copies = []
            for k, (src, dst) in enumerate(((s_loc, s_rx), (l_loc, l_rx))):
                c = pltpu.make_async_remote_copy(
                    src_ref=src, dst_ref=dst,
                    send_sem=send_sems.at[k], recv_sem=recv_sems.at[k],
                    device_id=partner,
                    device_id_type=pl.DeviceIdType.MESH)
                c.start()
                copies.append(c)
            for c in copies:
                c.wait()

            out_ref[...] = (jnp.log(s_loc[...] + s_rx[...])
                            - (l_loc[...] + l_rx[...]))

            pl.semaphore_signal(ack_sem, 1, device_id=partner,
                                device_id_type=pl.DeviceIdType.MESH)
            pl.semaphore_wait(ack_sem, 1)

    out = pl.pallas_call(
        body,
        grid=(NBLK,),
        in_specs=[
            pl.BlockSpec((T, D), lambda j: (0, 0)),
            pl.BlockSpec((D, BV), lambda j: (0, j)),
            pl.BlockSpec((T, 1), lambda j: (0, 0)),
        ],
        out_specs=pl.BlockSpec((T, 1), lambda j: (0, 0)),
        out_shape=jax.ShapeDtypeStruct((T, 1), jnp.float32),
        scratch_shapes=[
            pltpu.VMEM((T, BV), jnp.float32),
            pltpu.VMEM((T, BV), jnp.float32),
            pltpu.VMEM((T, 1), jnp.float32),
            pltpu.VMEM((T, 1), jnp.float32),
            pltpu.VMEM((T, 1), jnp.float32),
            pltpu.VMEM((T, 1), jnp.float32),
            pltpu.SemaphoreType.DMA((2,)),
            pltpu.SemaphoreType.DMA((2,)),
            pltpu.SemaphoreType.REGULAR,
        ],
        compiler_params=pltpu.CompilerParams(
            dimension_semantics=("arbitrary",),
            vmem_limit_bytes=64 * 1024 * 1024,
        ),
    )(x, W, labels2d)
    return out.reshape(T)


# device time: 196485 ns/iter; 1.1628x vs baseline; 1.1579x over previous
import jax
import jax.numpy as jnp
from jax import lax
from jax.experimental import pallas as pl
from jax.experimental.pallas import tpu as pltpu

T = 2048
D = 4096
V_SHARD = 8192
BV = 512
NBLK = V_SHARD // BV

_S, _L = 0, 1


def kernel(x, W, labels):
    labels2d = labels.reshape(T, 1)

    def body(x_ref, w_ref, lbl_ref, out_ref,
             stats, rx, send_sem, recv_sem, ack_sem):
        j = pl.program_id(0)
        my_x = lax.axis_index("x")
        my_y = lax.axis_index("y")
        my_z = lax.axis_index("z")

        logits = jnp.dot(x_ref[...], w_ref[...],
                         preferred_element_type=jnp.float32)

        offset = my_x * V_SHARD + j * BV
        idx = lbl_ref[...] - offset
        col = lax.broadcasted_iota(jnp.int32, (T, BV), 1)
        lval = jnp.sum(jnp.where(col == idx, logits, 0.0),
                       axis=1, keepdims=True)
        s = jnp.sum(jnp.exp(logits), axis=1, keepdims=True)

        @pl.when(j == 0)
        def _():
            stats[:, _S:_S + 1] = s
            stats[:, _L:_L + 1] = lval

        @pl.when(j > 0)
        def _():
            stats[:, _S:_S + 1] = stats[:, _S:_S + 1] + s
            stats[:, _L:_L + 1] = stats[:, _L:_L + 1] + lval

        @pl.when(j == NBLK - 1)
        def _():
            partner = (1 - my_x, my_y, my_z)
            rdma = pltpu.make_async_remote_copy(
                src_ref=stats, dst_ref=rx,
                send_sem=send_sem, recv_sem=recv_sem,
                device_id=partner,
                device_id_type=pl.DeviceIdType.MESH)
            rdma.start()
            rdma.wait()

            s_tot = stats[:, _S:_S + 1] + rx[:, _S:_S + 1]
            l_tot = stats[:, _L:_L + 1] + rx[:, _L:_L + 1]
            out_ref[...] = jnp.log(s_tot) - l_tot

            pl.semaphore_signal(ack_sem, 1, device_id=partner,
                                device_id_type=pl.DeviceIdType.MESH)
            pl.semaphore_wait(ack_sem, 1)

    out = pl.pallas_call(
        body,
        grid=(NBLK,),
        in_specs=[
            pl.BlockSpec((T, D), lambda j: (0, 0)),
            pl.BlockSpec((D, BV), lambda j: (0, j)),
            pl.BlockSpec((T, 1), lambda j: (0, 0)),
        ],
        out_specs=pl.BlockSpec((T, 1), lambda j: (0, 0)),
        out_shape=jax.ShapeDtypeStruct((T, 1), jnp.float32),
        scratch_shapes=[
            pltpu.VMEM((T, 128), jnp.float32),
            pltpu.VMEM((T, 128), jnp.float32),
            pltpu.SemaphoreType.DMA,
            pltpu.SemaphoreType.DMA,
            pltpu.SemaphoreType.REGULAR,
        ],
        compiler_params=pltpu.CompilerParams(
            dimension_semantics=("arbitrary",),
            vmem_limit_bytes=64 * 1024 * 1024,
        ),
    )(x, W, labels2d)
    return out.reshape(T)
